# baseline (speedup 1.0000x reference)
import functools
import jax
import jax.numpy as jnp
from jax.experimental import pallas as pl
from jax.experimental.pallas import tpu as pltpu

N = 2048
DIM = 768
NUM_HEADS = 12
KEY_DIM = 16
VALUE_DIM = 16
TOP_K = 64
IDX_HEADS = 2
IDX_DIM = 8
SCALE = KEY_DIM ** (-0.5)

BQ = 1024
GRID = N // BQ

_CONTRACT_T = (((1,), (1,)), ((), ()))
_CONTRACT = (((1,), (0,)), ((), ()))


def _dot(a, b, dims):
  return jax.lax.dot_general(a.astype(jnp.bfloat16), b.astype(jnp.bfloat16),
                             dims, preferred_element_type=jnp.float32)


def _sortable_keys(s):
  bits = jax.lax.bitcast_convert_type(s, jnp.int32)
  return jnp.where(bits >= 0, bits, bits ^ jnp.int32(0x7FFFFFFF))


def _row_count_ge(keys, thresh):
  return jnp.sum((keys >= thresh).astype(jnp.int32), axis=1, keepdims=True)


def _topk_mask(s):
  keys = _sortable_keys(s)
  k = jnp.int32(TOP_K)

  ge0 = _row_count_ge(keys, jnp.zeros((BQ, 1), jnp.int32)) >= k
  lo = jnp.where(ge0, jnp.int32(0), jnp.int32(-2147483648))
  hi = jnp.where(ge0, jnp.int32(2147483647), jnp.int32(-1))

  def body(_, carry):
    lo, hi = carry
    diff = hi - lo
    mid = lo + (diff >> 1) + (diff & 1)
    ge = _row_count_ge(keys, mid) >= k
    return jnp.where(ge, mid, lo), jnp.where(ge, hi, mid - 1)

  lo, hi = jax.lax.fori_loop(0, 31, body, (lo, hi))
  t = lo

  gt = keys > t
  eq = keys == t
  need = k - jnp.sum(gt.astype(jnp.int32), axis=1, keepdims=True)
  iota = jax.lax.broadcasted_iota(jnp.int32, (BQ, N), 1)
  lo2 = jnp.zeros((BQ, 1), jnp.int32)
  hi2 = jnp.full((BQ, 1), N - 1, jnp.int32)

  def body2(_, carry):
    lo2, hi2 = carry
    mid = (lo2 + hi2) >> 1
    cnt = jnp.sum((eq & (iota <= mid)).astype(jnp.int32), axis=1,
                  keepdims=True)
    ge = cnt >= need
    return jnp.where(ge, lo2, mid + 1), jnp.where(ge, mid, hi2)

  lo2, hi2 = jax.lax.fori_loop(0, 11, body2, (lo2, hi2))
  return gt | (eq & (iota <= lo2))


def _body(xb_ref, x_ref, wq_ref, wk_ref, wv_ref, wo_ref,
          wqi_ref, wki_ref, wwi_ref, out_ref, ki_scr, k_scr, v_scr):
  i = pl.program_id(0)

  @pl.when(i == 0)
  def _():
    x = x_ref[...]
    ki_scr[...] = _dot(x, wki_ref[...], _CONTRACT).astype(jnp.bfloat16)
    k_scr[...] = _dot(x, wk_ref[...], _CONTRACT).astype(jnp.bfloat16)
    v_scr[...] = _dot(x, wv_ref[...], _CONTRACT).astype(jnp.bfloat16)

  xb = xb_ref[...]

  qi = _dot(xb, wqi_ref[...], _CONTRACT).astype(jnp.bfloat16)
  wi = _dot(xb, wwi_ref[...], _CONTRACT).astype(jnp.bfloat16)
  wi = wi.astype(jnp.float32)
  ki = ki_scr[...]
  s = jnp.zeros((BQ, N), jnp.float32)
  for h in range(IDX_HEADS):
    qih = qi[:, h * IDX_DIM:(h + 1) * IDX_DIM]
    sh = jnp.maximum(_dot(qih, ki, _CONTRACT_T), 0.0).astype(jnp.bfloat16)
    s = s + wi[:, h:h + 1] * sh.astype(jnp.float32)

  mask = _topk_mask(s)

  qb = _dot(xb, wq_ref[...], _CONTRACT).astype(jnp.bfloat16)
  kk = k_scr[...]
  vv_ext = jnp.concatenate(
      [v_scr[...], jnp.ones((N, 1), jnp.bfloat16)], axis=1)
  acc = jnp.zeros((BQ, DIM), jnp.float32)
  for h in range(NUM_HEADS):
    qh = qb[:, h * KEY_DIM:(h + 1) * KEY_DIM]
    sc = _dot(qh, kk, _CONTRACT_T) * SCALE
    sc = jnp.where(mask, sc, -jnp.inf)
    m = jnp.max(sc, axis=1, keepdims=True)
    p = jnp.exp(sc - m)
    pv = _dot(p, vv_ext, _CONTRACT)
    out_h = pv[:, :VALUE_DIM] / pv[:, VALUE_DIM:]
    wo_h = wo_ref[h * VALUE_DIM:(h + 1) * VALUE_DIM, :]
    acc = acc + _dot(out_h, wo_h, _CONTRACT)
  out_ref[...] = acc


@jax.jit
def kernel(x, Wq, Wk, Wv, Wo, Wq_idx, Wk_idx, Ww_idx):
  b, n, d = x.shape
  x2 = x.reshape(n, d)
  out = pl.pallas_call(
      _body,
      grid=(GRID,),
      in_specs=[
          pl.BlockSpec((BQ, DIM), lambda i: (i, 0)),
          pl.BlockSpec((N, DIM), lambda i: (0, 0)),
          pl.BlockSpec((DIM, NUM_HEADS * KEY_DIM), lambda i: (0, 0)),
          pl.BlockSpec((DIM, KEY_DIM), lambda i: (0, 0)),
          pl.BlockSpec((DIM, VALUE_DIM), lambda i: (0, 0)),
          pl.BlockSpec((NUM_HEADS * VALUE_DIM, DIM), lambda i: (0, 0)),
          pl.BlockSpec((DIM, IDX_HEADS * IDX_DIM), lambda i: (0, 0)),
          pl.BlockSpec((DIM, IDX_DIM), lambda i: (0, 0)),
          pl.BlockSpec((DIM, IDX_HEADS), lambda i: (0, 0)),
      ],
      out_specs=pl.BlockSpec((BQ, DIM), lambda i: (i, 0)),
      out_shape=jax.ShapeDtypeStruct((N, DIM), jnp.float32),
      scratch_shapes=[
          pltpu.VMEM((N, IDX_DIM), jnp.bfloat16),
          pltpu.VMEM((N, KEY_DIM), jnp.bfloat16),
          pltpu.VMEM((N, VALUE_DIM), jnp.bfloat16),
      ],
  )(x2, x2, Wq, Wk, Wv, Wo, Wq_idx, Wk_idx, Ww_idx)
  return out.reshape(b, n, d)


# speedup vs baseline: 1.1230x; 1.1230x over previous
import functools
import jax
import jax.numpy as jnp
from jax.experimental import pallas as pl
from jax.experimental.pallas import tpu as pltpu

N = 2048
DIM = 768
NUM_HEADS = 12
KEY_DIM = 16
VALUE_DIM = 16
TOP_K = 64
IDX_HEADS = 2
IDX_DIM = 8
SCALE = KEY_DIM ** (-0.5)

BQ = 512
GRID = N // BQ

_CONTRACT_T = (((1,), (1,)), ((), ()))
_CONTRACT = (((1,), (0,)), ((), ()))


def _dot(a, b, dims):
  return jax.lax.dot_general(a.astype(jnp.bfloat16), b.astype(jnp.bfloat16),
                             dims, preferred_element_type=jnp.float32)


def _sortable_keys(s):
  bits = jax.lax.bitcast_convert_type(s, jnp.int32)
  return jnp.where(bits >= 0, bits, bits ^ jnp.int32(0x7FFFFFFF))


def _row_count_ge(keys, thresh):
  return jnp.sum((keys >= thresh).astype(jnp.int32), axis=1, keepdims=True)


def _topk_mask(s):
  keys = _sortable_keys(s)
  k = jnp.int32(TOP_K)

  ge0 = _row_count_ge(keys, jnp.zeros((BQ, 1), jnp.int32)) >= k
  lo = jnp.where(ge0, jnp.int32(0), jnp.int32(-2147483648))
  hi = jnp.where(ge0, jnp.int32(2147483647), jnp.int32(-1))

  def body(_, carry):
    lo, hi = carry
    diff = hi - lo
    mid = lo + (diff >> 1) + (diff & 1)
    ge = _row_count_ge(keys, mid) >= k
    return jnp.where(ge, mid, lo), jnp.where(ge, hi, mid - 1)

  lo, hi = jax.lax.fori_loop(0, 31, body, (lo, hi))
  t = lo

  gt = keys > t
  eq = keys == t
  need = k - jnp.sum(gt.astype(jnp.int32), axis=1, keepdims=True)
  iota = jax.lax.broadcasted_iota(jnp.int32, (BQ, N), 1)
  lo2 = jnp.zeros((BQ, 1), jnp.int32)
  hi2 = jnp.full((BQ, 1), N - 1, jnp.int32)

  def body2(_, carry):
    lo2, hi2 = carry
    mid = (lo2 + hi2) >> 1
    cnt = jnp.sum((eq & (iota <= mid)).astype(jnp.int32), axis=1,
                  keepdims=True)
    ge = cnt >= need
    return jnp.where(ge, lo2, mid + 1), jnp.where(ge, mid, hi2)

  lo2, hi2 = jax.lax.fori_loop(0, 11, body2, (lo2, hi2))
  return gt | (eq & (iota <= lo2))


def _body(xb_ref, x_ref, wq_ref, wk_ref, wv_ref, wo_ref,
          wqi_ref, wki_ref, wwi_ref, out_ref, ki_scr, k_scr, v_scr):
  i = pl.program_id(0)

  @pl.when(i == 0)
  def _():
    x = x_ref[...]
    ki_scr[...] = _dot(x, wki_ref[...], _CONTRACT).astype(jnp.bfloat16)
    k_scr[...] = _dot(x, wk_ref[...], _CONTRACT).astype(jnp.bfloat16)
    v_scr[...] = _dot(x, wv_ref[...], _CONTRACT).astype(jnp.bfloat16)

  xb = xb_ref[...]

  qi = _dot(xb, wqi_ref[...], _CONTRACT).astype(jnp.bfloat16)
  wi = _dot(xb, wwi_ref[...], _CONTRACT).astype(jnp.bfloat16)
  wi = wi.astype(jnp.float32)
  ki = ki_scr[...]
  s = jnp.zeros((BQ, N), jnp.float32)
  for h in range(IDX_HEADS):
    qih = qi[:, h * IDX_DIM:(h + 1) * IDX_DIM]
    sh = jnp.maximum(_dot(qih, ki, _CONTRACT_T), 0.0).astype(jnp.bfloat16)
    s = s + wi[:, h:h + 1] * sh.astype(jnp.float32)

  mask = _topk_mask(s)

  qb = _dot(xb, wq_ref[...], _CONTRACT).astype(jnp.bfloat16)
  kk = k_scr[...]
  vv_ext = jnp.concatenate(
      [v_scr[...], jnp.ones((N, 1), jnp.bfloat16)], axis=1)
  acc = jnp.zeros((BQ, DIM), jnp.float32)
  neg = jnp.float32(-jnp.inf)
  for h in range(NUM_HEADS):
    qh = qb[:, h * KEY_DIM:(h + 1) * KEY_DIM]
    sc = _dot(qh, kk, _CONTRACT_T) * SCALE
    sc = jnp.where(mask, sc, neg).astype(jnp.bfloat16)
    m = jnp.max(sc, axis=1, keepdims=True)
    p = jnp.exp((sc - m).astype(jnp.float32)).astype(jnp.bfloat16)
    pv = _dot(p, vv_ext, _CONTRACT)
    out_h = pv[:, :VALUE_DIM] / pv[:, VALUE_DIM:]
    wo_h = wo_ref[h * VALUE_DIM:(h + 1) * VALUE_DIM, :]
    acc = acc + _dot(out_h, wo_h, _CONTRACT)
  out_ref[...] = acc


@jax.jit
def kernel(x, Wq, Wk, Wv, Wo, Wq_idx, Wk_idx, Ww_idx):
  b, n, d = x.shape
  x2 = x.reshape(n, d)
  out = pl.pallas_call(
      _body,
      grid=(GRID,),
      in_specs=[
          pl.BlockSpec((BQ, DIM), lambda i: (i, 0)),
          pl.BlockSpec((N, DIM), lambda i: (0, 0)),
          pl.BlockSpec((DIM, NUM_HEADS * KEY_DIM), lambda i: (0, 0)),
          pl.BlockSpec((DIM, KEY_DIM), lambda i: (0, 0)),
          pl.BlockSpec((DIM, VALUE_DIM), lambda i: (0, 0)),
          pl.BlockSpec((NUM_HEADS * VALUE_DIM, DIM), lambda i: (0, 0)),
          pl.BlockSpec((DIM, IDX_HEADS * IDX_DIM), lambda i: (0, 0)),
          pl.BlockSpec((DIM, IDX_DIM), lambda i: (0, 0)),
          pl.BlockSpec((DIM, IDX_HEADS), lambda i: (0, 0)),
      ],
      out_specs=pl.BlockSpec((BQ, DIM), lambda i: (i, 0)),
      out_shape=jax.ShapeDtypeStruct((N, DIM), jnp.float32),
      scratch_shapes=[
          pltpu.VMEM((N, IDX_DIM), jnp.bfloat16),
          pltpu.VMEM((N, KEY_DIM), jnp.bfloat16),
          pltpu.VMEM((N, VALUE_DIM), jnp.bfloat16),
      ],
  )(x2, x2, Wq, Wk, Wv, Wo, Wq_idx, Wk_idx, Ww_idx)
  return out.reshape(b, n, d)


# speedup vs baseline: 1.1558x; 1.0292x over previous
import functools
import jax
import jax.numpy as jnp
from jax.experimental import pallas as pl
from jax.experimental.pallas import tpu as pltpu

N = 2048
DIM = 768
NUM_HEADS = 12
KEY_DIM = 16
VALUE_DIM = 16
TOP_K = 64
IDX_HEADS = 2
IDX_DIM = 8
SCALE = KEY_DIM ** (-0.5)

BQ = 512
GRID = N // BQ

_CONTRACT_T = (((1,), (1,)), ((), ()))
_CONTRACT = (((1,), (0,)), ((), ()))


def _dot(a, b, dims):
  return jax.lax.dot_general(a.astype(jnp.bfloat16), b.astype(jnp.bfloat16),
                             dims, preferred_element_type=jnp.float32)


def _sortable_keys(s):
  bits = jax.lax.bitcast_convert_type(s, jnp.int32)
  return jnp.where(bits >= 0, bits, bits ^ jnp.int32(0x7FFFFFFF))


def _row_count_ge(keys, thresh):
  return jnp.sum((keys >= thresh).astype(jnp.int32), axis=1, keepdims=True)


def _topk_mask(s):
  keys = _sortable_keys(s)
  k = jnp.int32(TOP_K)

  ge0 = _row_count_ge(keys, jnp.zeros((BQ, 1), jnp.int32)) >= k
  lo = jnp.where(ge0, jnp.int32(0), jnp.int32(-2147483648))
  hi = jnp.where(ge0, jnp.int32(2147483647), jnp.int32(-1))

  def body(_, carry):
    lo, hi = carry
    diff = hi - lo
    mid = lo + (diff >> 1) + (diff & 1)
    ge = _row_count_ge(keys, mid) >= k
    return jnp.where(ge, mid, lo), jnp.where(ge, hi, mid - 1)

  lo, hi = jax.lax.fori_loop(0, 31, body, (lo, hi))
  t = lo

  gt = keys > t
  eq = keys == t
  need = k - jnp.sum(gt.astype(jnp.int32), axis=1, keepdims=True)
  iota = jax.lax.broadcasted_iota(jnp.int32, (BQ, N), 1)
  ii = jnp.where(eq, iota, jnp.int32(N))
  lo2 = jnp.zeros((BQ, 1), jnp.int32)
  hi2 = jnp.full((BQ, 1), N - 1, jnp.int32)

  def body2(_, carry):
    lo2, hi2 = carry
    mid = (lo2 + hi2) >> 1
    cnt = jnp.sum((ii <= mid).astype(jnp.int32), axis=1, keepdims=True)
    ge = cnt >= need
    return jnp.where(ge, lo2, mid + 1), jnp.where(ge, mid, hi2)

  lo2, hi2 = jax.lax.fori_loop(0, 11, body2, (lo2, hi2))
  return gt | (ii <= lo2)


def _body(xb_ref, x_ref, wq_ref, wk_ref, wv_ref, wo_ref,
          wqi_ref, wki_ref, wwi_ref, out_ref, ki_scr, k_scr, v_scr):
  i = pl.program_id(0)

  @pl.when(i == 0)
  def _():
    x = x_ref[...]
    ki_scr[...] = _dot(x, wki_ref[...], _CONTRACT).astype(jnp.bfloat16)
    k_scr[...] = _dot(x, wk_ref[...], _CONTRACT).astype(jnp.bfloat16)
    v_scr[...] = _dot(x, wv_ref[...], _CONTRACT).astype(jnp.bfloat16)

  xb = xb_ref[...]

  qi = _dot(xb, wqi_ref[...], _CONTRACT).astype(jnp.bfloat16)
  wi = _dot(xb, wwi_ref[...], _CONTRACT).astype(jnp.bfloat16)
  wi = wi.astype(jnp.float32)
  ki = ki_scr[...]
  s = jnp.zeros((BQ, N), jnp.float32)
  for h in range(IDX_HEADS):
    qih = qi[:, h * IDX_DIM:(h + 1) * IDX_DIM]
    sh = jnp.maximum(_dot(qih, ki, _CONTRACT_T), 0.0).astype(jnp.bfloat16)
    s = s + wi[:, h:h + 1] * sh.astype(jnp.float32)

  mask = _topk_mask(s)

  qb = _dot(xb, wq_ref[...], _CONTRACT).astype(jnp.bfloat16)
  kk = k_scr[...]
  vv_ext = jnp.concatenate(
      [v_scr[...], jnp.ones((N, 1), jnp.bfloat16)], axis=1)
  acc = jnp.zeros((BQ, DIM), jnp.float32)
  neg = jnp.float32(-jnp.inf)
  for h in range(NUM_HEADS):
    qh = qb[:, h * KEY_DIM:(h + 1) * KEY_DIM]
    sc = _dot(qh, kk, _CONTRACT_T) * SCALE
    sc = jnp.where(mask, sc, neg).astype(jnp.bfloat16)
    m = jnp.max(sc, axis=1, keepdims=True)
    p = jnp.exp((sc - m).astype(jnp.float32)).astype(jnp.bfloat16)
    pv = _dot(p, vv_ext, _CONTRACT)
    out_h = pv[:, :VALUE_DIM] / pv[:, VALUE_DIM:]
    wo_h = wo_ref[h * VALUE_DIM:(h + 1) * VALUE_DIM, :]
    acc = acc + _dot(out_h, wo_h, _CONTRACT)
  out_ref[...] = acc


@jax.jit
def kernel(x, Wq, Wk, Wv, Wo, Wq_idx, Wk_idx, Ww_idx):
  b, n, d = x.shape
  x2 = x.reshape(n, d)
  out = pl.pallas_call(
      _body,
      grid=(GRID,),
      in_specs=[
          pl.BlockSpec((BQ, DIM), lambda i: (i, 0)),
          pl.BlockSpec((N, DIM), lambda i: (0, 0)),
          pl.BlockSpec((DIM, NUM_HEADS * KEY_DIM), lambda i: (0, 0)),
          pl.BlockSpec((DIM, KEY_DIM), lambda i: (0, 0)),
          pl.BlockSpec((DIM, VALUE_DIM), lambda i: (0, 0)),
          pl.BlockSpec((NUM_HEADS * VALUE_DIM, DIM), lambda i: (0, 0)),
          pl.BlockSpec((DIM, IDX_HEADS * IDX_DIM), lambda i: (0, 0)),
          pl.BlockSpec((DIM, IDX_DIM), lambda i: (0, 0)),
          pl.BlockSpec((DIM, IDX_HEADS), lambda i: (0, 0)),
      ],
      out_specs=pl.BlockSpec((BQ, DIM), lambda i: (i, 0)),
      out_shape=jax.ShapeDtypeStruct((N, DIM), jnp.float32),
      scratch_shapes=[
          pltpu.VMEM((N, IDX_DIM), jnp.bfloat16),
          pltpu.VMEM((N, KEY_DIM), jnp.bfloat16),
          pltpu.VMEM((N, VALUE_DIM), jnp.bfloat16),
      ],
  )(x2, x2, Wq, Wk, Wv, Wo, Wq_idx, Wk_idx, Ww_idx)
  return out.reshape(b, n, d)
